# baseline (speedup 1.0000x reference)
import functools

import jax
import jax.numpy as jnp
from jax import lax
from jax.experimental import pallas as pl
from jax.experimental.pallas import tpu as pltpu
from jax.experimental.pallas import tpu_sc as plsc

N = 10000
E = 320000
F = 128
_LOG2 = 0.6931471805599453

NC = 2
NS = 16
NW = NC * NS
EPW = E // NW
C = 80
NCHUNK = EPW // C
N_PAD = 10240
RPT = N_PAD // NS
ZR = 128
NZ = RPT // ZR
E_PAD = 3840


def _ssp(x):
    return jax.nn.softplus(x) - _LOG2


def _node_dense(node, W1):
    BN = 1000

    def body(n_ref, w_ref, o_ref):
        o_ref[...] = jnp.dot(n_ref[...], w_ref[...],
                             preferred_element_type=jnp.float32)

    return pl.pallas_call(
        body,
        grid=(N // BN,),
        in_specs=[pl.BlockSpec((BN, F), lambda i: (i, 0)),
                  pl.BlockSpec((F, F), lambda i: (0, 0))],
        out_specs=pl.BlockSpec((BN, F), lambda i: (i, 0)),
        out_shape=jax.ShapeDtypeStruct((N, F), jnp.float32),
    )(node, W1)


def _edge_mlp(edge, We1, be1, We2, be2, nrows, row0):
    BE = 640
    blk0 = row0 // BE
    nreal = (E // 2) // BE

    def body(e_ref, w1_ref, b1_ref, w2_ref, b2_ref, o_ref):
        @pl.when(pl.program_id(0) < nreal)
        def _():
            t = jnp.dot(e_ref[...], w1_ref[...],
                        preferred_element_type=jnp.float32)
            t = _ssp(t + b1_ref[...])
            o_ref[...] = (jnp.dot(t, w2_ref[...],
                                  preferred_element_type=jnp.float32)
                          + b2_ref[...]).astype(jnp.bfloat16)
        @pl.when(pl.program_id(0) >= nreal)
        def _():
            o_ref[...] = jnp.zeros((BE, F), jnp.bfloat16)

    return pl.pallas_call(
        body,
        grid=(nrows // BE,),
        in_specs=[pl.BlockSpec((BE, F),
                               lambda i: (jnp.minimum(i, nreal - 1) + blk0, 0)),
                  pl.BlockSpec((F, F), lambda i: (0, 0)),
                  pl.BlockSpec((1, F), lambda i: (0, 0)),
                  pl.BlockSpec((F, F), lambda i: (0, 0)),
                  pl.BlockSpec((1, F), lambda i: (0, 0))],
        out_specs=pl.BlockSpec((BE, F), lambda i: (i, 0)),
        out_shape=jax.ShapeDtypeStruct((nrows, F), jnp.bfloat16),
    )(edge, We1, be1.reshape(1, F), We2, be2.reshape(1, F))


def _out_mlp(node, p_a, p_b, W2, b2, W3, b3):
    BN = 1000

    def body(n_ref, p_ref, q_ref, w2_ref, b2_ref, w3_ref, b3_ref, o_ref):
        p = p_ref[0] + p_ref[1] + q_ref[0] + q_ref[1]
        t = _ssp(jnp.dot(p, w2_ref[...],
                         preferred_element_type=jnp.float32) + b2_ref[...])
        o_ref[...] = (n_ref[...] +
                      jnp.dot(t, w3_ref[...],
                              preferred_element_type=jnp.float32) + b3_ref[...])

    return pl.pallas_call(
        body,
        grid=(N // BN,),
        in_specs=[pl.BlockSpec((BN, F), lambda i: (i, 0)),
                  pl.BlockSpec((NC, BN, F), lambda i: (0, i, 0)),
                  pl.BlockSpec((NC, BN, F), lambda i: (0, i, 0)),
                  pl.BlockSpec((F, F), lambda i: (0, 0)),
                  pl.BlockSpec((1, F), lambda i: (0, 0)),
                  pl.BlockSpec((F, F), lambda i: (0, 0)),
                  pl.BlockSpec((1, F), lambda i: (0, 0))],
        out_specs=pl.BlockSpec((BN, F), lambda i: (i, 0)),
        out_shape=jax.ShapeDtypeStruct((N, F), jnp.float32),
    )(node, p_a, p_b, W2, b2.reshape(1, F), W3, b3.reshape(1, F))


def _gather_mul_scatter(x, h, srca, dsta, zrows, epw, c):
    nchunk = epw // c
    assert nchunk >= 4 and nchunk % 2 == 0 and c % 16 == 0 and nchunk * c == epw
    mesh = plsc.VectorSubcoreMesh(core_axis_name="c", subcore_axis_name="s")

    @functools.partial(
        pl.kernel,
        out_type=jax.ShapeDtypeStruct((NC, N_PAD, F), jnp.float32),
        mesh=mesh,
        scratch_types=[
            [pltpu.VMEM((c,), jnp.int32)] * 2,
            [pltpu.VMEM((c,), jnp.int32)] * 2,
            [pltpu.VMEM((c, F), jnp.float32)] * 2,
            [pltpu.VMEM((c, F // 2), jnp.int32)] * 2,
            pltpu.VMEM_SHARED((N_PAD, F), jnp.float32),
            [pltpu.SemaphoreType.DMA] * 2,
            [pltpu.SemaphoreType.DMA] * 2,
            [pltpu.SemaphoreType.DMA] * 2,
            [pltpu.SemaphoreType.DMA] * 2,
        ],
    )
    def run(x_hbm, h_hbm, src_hbm, dst_hbm, z_hbm, out_hbm,
            srcv, dstv, xr, hr, acc, sem_si, sem_di, sem_gh, sem_s):
        cid = lax.axis_index("c")
        sid = lax.axis_index("s")

        for j in range(NZ):
            pltpu.sync_copy(z_hbm, acc.at[pl.ds(sid * RPT + j * ZR, ZR)])
        plsc.subcore_barrier()

        base = (cid * NS + sid) * epw

        def src_start(i, b):
            pltpu.async_copy(src_hbm.at[pl.ds(base + i * c, c)],
                             srcv[b], sem_si[b])

        def dst_start(i, b):
            pltpu.async_copy(dst_hbm.at[pl.ds(base + i * c, c)],
                             dstv[b], sem_di[b])

        def gh_start(i, b):
            pltpu.async_copy(x_hbm.at[srcv[b]], xr[b], sem_gh[b])
            pltpu.async_copy(h_hbm.at[pl.ds(base + i * c, c)], hr[b],
                             sem_gh[b])

        def gh_wait(b):
            pltpu.make_async_copy(x_hbm.at[srcv[b]], xr[b], sem_gh[b]).wait()
            pltpu.make_async_copy(h_hbm.at[pl.ds(base, c)], hr[b],
                                  sem_gh[b]).wait()

        def si_wait(b):
            pltpu.make_async_copy(src_hbm.at[pl.ds(base, c)], srcv[b],
                                  sem_si[b]).wait()

        def di_wait(b):
            pltpu.make_async_copy(dst_hbm.at[pl.ds(base, c)], dstv[b],
                                  sem_di[b]).wait()

        def s_start(b):
            pltpu.async_copy(xr[b], acc.at[dstv[b]], sem_s[b], add=True)

        def s_wait(b):
            pltpu.make_async_copy(xr[b], acc.at[dstv[b]], sem_s[b]).wait()

        def mul(b):
            def mrow(j, c2):
                for u in range(4):
                    r = j * 4 + u
                    for m in range(F // 32):
                        hw = hr[b][r, pl.ds(m * 16, 16)]
                        lo = lax.bitcast_convert_type(hw << 16,
                                                      jnp.float32)
                        hi = lax.bitcast_convert_type(
                            hw & jnp.int32(-65536), jnp.float32)
                        sl0 = pl.ds(m * 32, 16)
                        sl1 = pl.ds(m * 32 + 16, 16)
                        xr[b][r, sl0] = xr[b][r, sl0] * lo
                        xr[b][r, sl1] = xr[b][r, sl1] * hi
                return c2
            lax.fori_loop(0, c // 4, mrow, 0)

        def maybe(cond, fn):
            if isinstance(cond, bool):
                if cond:
                    fn()
            else:
                pl.when(cond)(fn)

        def emit_step(i, cur, first):
            nxt = 1 - cur
            gh_wait(cur)
            maybe(i + 2 < nchunk, lambda: src_start(i + 2, cur))
            if not first:
                s_wait(nxt)

            def nx():
                dst_start(i + 1, nxt)
                si_wait(nxt)
                gh_start(i + 1, nxt)
            maybe(i + 1 < nchunk, nx)
            mul(cur)
            di_wait(cur)
            s_start(cur)

        src_start(0, 0)
        dst_start(0, 0)
        src_start(1, 1)
        si_wait(0)
        gh_start(0, 0)

        emit_step(0, 0, first=True)
        emit_step(1, 1, first=False)

        def pair(j, carry):
            i = j * 2
            emit_step(i, 0, first=False)
            emit_step(i + 1, 1, first=False)
            return carry

        lax.fori_loop(1, nchunk // 2, pair, 0)
        s_wait((nchunk - 1) % 2)

        plsc.subcore_barrier()
        pltpu.sync_copy(acc.at[pl.ds(sid * RPT, RPT)],
                        out_hbm.at[cid, pl.ds(sid * RPT, RPT)])

    return run(x, h, srca, dsta, zrows)


def _interleave_perm():
    import numpy as _np
    tau = _np.empty((F,), dtype=_np.int32)
    for m in range(F // 32):
        for t in range(16):
            tau[32 * m + t] = 32 * m + 2 * t
            tau[32 * m + 16 + t] = 32 * m + 2 * t + 1
    return tau


_TAU = _interleave_perm()


def kernel(node, edge, edge_index, W1, We1, be1, We2, be2, W2, b2, W3, b3):
    ei = edge_index.astype(jnp.int32)
    tau = jnp.asarray(_TAU)
    x = _node_dense(node, W1[:, tau])
    zrows = jnp.zeros((ZR, F), jnp.float32)
    EH = E // 2
    EHP = EH + E_PAD
    pad_dst = N + (jnp.arange(E_PAD, dtype=jnp.int32) % (N_PAD - N))
    pad_src = jnp.arange(E_PAD, dtype=jnp.int32) % N
    src_a = jnp.concatenate([ei[1, :EH], pad_src])
    dst_a = jnp.concatenate([ei[0, :EH], pad_dst])
    src_b = jnp.concatenate([ei[1, EH:], pad_src])
    dst_b = jnp.concatenate([ei[0, EH:], pad_dst])

    def packed(h):
        return jax.lax.bitcast_convert_type(
            h.reshape(EHP, F // 2, 2), jnp.int32)

    h_a = packed(_edge_mlp(edge, We1, be1, We2, be2, EHP, 0))
    p_a = _gather_mul_scatter(x, h_a, src_a, dst_a, zrows, EHP // NW, 80)
    h_b = packed(_edge_mlp(edge, We1, be1, We2, be2, EHP, EH))
    p_b = _gather_mul_scatter(x, h_b, src_b, dst_b, zrows, EHP // NW, 80)
    return _out_mlp(node, p_a, p_b, W2[tau, :], b2, W3, b3)


# speedup vs baseline: 2.3511x; 2.3511x over previous
import functools

import jax
import jax.numpy as jnp
from jax import lax
from jax.experimental import pallas as pl
from jax.experimental.pallas import tpu as pltpu
from jax.experimental.pallas import tpu_sc as plsc

N = 10000
E = 320000
F = 128
_LOG2 = 0.6931471805599453

NC = 2
NS = 16
NW = NC * NS
EPW = E // NW
C = 80
NCHUNK = EPW // C
N_PAD = 10240
RPT = N_PAD // NS
ZR = 128
NZ = RPT // ZR
E_PAD = 3840


def _ssp(x):
    return jax.nn.softplus(x) - _LOG2


def _node_dense(node, W1):
    BN = 1000

    def body(n_ref, w_ref, o_ref):
        o_ref[...] = jnp.dot(n_ref[...], w_ref[...],
                             preferred_element_type=jnp.float32)

    return pl.pallas_call(
        body,
        grid=(N // BN,),
        in_specs=[pl.BlockSpec((BN, F), lambda i: (i, 0)),
                  pl.BlockSpec((F, F), lambda i: (0, 0))],
        out_specs=pl.BlockSpec((BN, F), lambda i: (i, 0)),
        out_shape=jax.ShapeDtypeStruct((N, F), jnp.float32),
    )(node, W1)


def _edge_mlp(edge, We1, be1, We2, be2, nrows, row0):
    BE = 640
    blk0 = row0 // BE
    nreal = (E // 2) // BE

    def body(e_ref, w1_ref, b1_ref, w2_ref, b2_ref, o_ref):
        @pl.when(pl.program_id(0) < nreal)
        def _():
            t = jnp.dot(e_ref[...], w1_ref[...],
                        preferred_element_type=jnp.float32)
            t = _ssp(t + b1_ref[...])
            o_ref[...] = jnp.dot(t, w2_ref[...],
                                 preferred_element_type=jnp.float32) + b2_ref[...]
        @pl.when(pl.program_id(0) >= nreal)
        def _():
            o_ref[...] = jnp.zeros((BE, F), jnp.float32)

    return pl.pallas_call(
        body,
        grid=(nrows // BE,),
        in_specs=[pl.BlockSpec((BE, F),
                               lambda i: (jnp.minimum(i, nreal - 1) + blk0, 0)),
                  pl.BlockSpec((F, F), lambda i: (0, 0)),
                  pl.BlockSpec((1, F), lambda i: (0, 0)),
                  pl.BlockSpec((F, F), lambda i: (0, 0)),
                  pl.BlockSpec((1, F), lambda i: (0, 0))],
        out_specs=pl.BlockSpec((BE, F), lambda i: (i, 0)),
        out_shape=jax.ShapeDtypeStruct((nrows, F), jnp.float32),
    )(edge, We1, be1.reshape(1, F), We2, be2.reshape(1, F))


def _out_mlp(node, p_a, p_b, W2, b2, W3, b3):
    BN = 1000

    def body(n_ref, p_ref, q_ref, w2_ref, b2_ref, w3_ref, b3_ref, o_ref):
        p = p_ref[0] + p_ref[1] + q_ref[0] + q_ref[1]
        t = _ssp(jnp.dot(p, w2_ref[...],
                         preferred_element_type=jnp.float32) + b2_ref[...])
        o_ref[...] = (n_ref[...] +
                      jnp.dot(t, w3_ref[...],
                              preferred_element_type=jnp.float32) + b3_ref[...])

    return pl.pallas_call(
        body,
        grid=(N // BN,),
        in_specs=[pl.BlockSpec((BN, F), lambda i: (i, 0)),
                  pl.BlockSpec((NC, BN, F), lambda i: (0, i, 0)),
                  pl.BlockSpec((NC, BN, F), lambda i: (0, i, 0)),
                  pl.BlockSpec((F, F), lambda i: (0, 0)),
                  pl.BlockSpec((1, F), lambda i: (0, 0)),
                  pl.BlockSpec((F, F), lambda i: (0, 0)),
                  pl.BlockSpec((1, F), lambda i: (0, 0))],
        out_specs=pl.BlockSpec((BN, F), lambda i: (i, 0)),
        out_shape=jax.ShapeDtypeStruct((N, F), jnp.float32),
    )(node, p_a, p_b, W2, b2.reshape(1, F), W3, b3.reshape(1, F))


def _gather_mul_scatter(x, h, srca, dsta, zrows, epw, c):
    nchunk = epw // c
    assert nchunk >= 4 and nchunk % 2 == 0 and c % 16 == 0 and nchunk * c == epw
    mesh = plsc.VectorSubcoreMesh(core_axis_name="c", subcore_axis_name="s")

    @functools.partial(
        pl.kernel,
        out_type=jax.ShapeDtypeStruct((NC, N_PAD, F), jnp.float32),
        mesh=mesh,
        scratch_types=[
            [pltpu.VMEM((c,), jnp.int32)] * 2,
            [pltpu.VMEM((c,), jnp.int32)] * 2,
            [pltpu.VMEM((c, F), jnp.float32)] * 2,
            [pltpu.VMEM((c, F), jnp.float32)] * 2,
            pltpu.VMEM_SHARED((N_PAD, F), jnp.float32),
            [pltpu.SemaphoreType.DMA] * 2,
            [pltpu.SemaphoreType.DMA] * 2,
            [pltpu.SemaphoreType.DMA] * 2,
            [pltpu.SemaphoreType.DMA] * 2,
        ],
    )
    def run(x_hbm, h_hbm, src_hbm, dst_hbm, z_hbm, out_hbm,
            srcv, dstv, xr, hr, acc, sem_si, sem_di, sem_gh, sem_s):
        cid = lax.axis_index("c")
        sid = lax.axis_index("s")

        for j in range(NZ):
            pltpu.sync_copy(z_hbm, acc.at[pl.ds(sid * RPT + j * ZR, ZR)])
        plsc.subcore_barrier()

        base = (cid * NS + sid) * epw

        def src_start(i, b):
            pltpu.async_copy(src_hbm.at[pl.ds(base + i * c, c)],
                             srcv[b], sem_si[b])

        def dst_start(i, b):
            pltpu.async_copy(dst_hbm.at[pl.ds(base + i * c, c)],
                             dstv[b], sem_di[b])

        def gh_start(i, b):
            pltpu.async_copy(x_hbm.at[srcv[b]], xr[b], sem_gh[b])
            pltpu.async_copy(h_hbm.at[pl.ds(base + i * c, c)], hr[b],
                             sem_gh[b])

        def gh_wait(b):
            pltpu.make_async_copy(x_hbm.at[srcv[b]], xr[b], sem_gh[b]).wait()
            pltpu.make_async_copy(h_hbm.at[pl.ds(base, c)], hr[b],
                                  sem_gh[b]).wait()

        def si_wait(b):
            pltpu.make_async_copy(src_hbm.at[pl.ds(base, c)], srcv[b],
                                  sem_si[b]).wait()

        def di_wait(b):
            pltpu.make_async_copy(dst_hbm.at[pl.ds(base, c)], dstv[b],
                                  sem_di[b]).wait()

        def s_start(b):
            pltpu.async_copy(xr[b], acc.at[dstv[b]], sem_s[b], add=True)

        def s_wait(b):
            pltpu.make_async_copy(xr[b], acc.at[dstv[b]], sem_s[b]).wait()

        def mul(b):
            def mrow(j, c2):
                for u in range(4):
                    r = j * 4 + u
                    for k in range(F // 16):
                        sl = pl.ds(k * 16, 16)
                        xr[b][r, sl] = xr[b][r, sl] * hr[b][r, sl]
                return c2
            lax.fori_loop(0, c // 4, mrow, 0)

        def maybe(cond, fn):
            if isinstance(cond, bool):
                if cond:
                    fn()
            else:
                pl.when(cond)(fn)

        def emit_step(i, cur, first):
            nxt = 1 - cur
            gh_wait(cur)
            maybe(i + 2 < nchunk, lambda: src_start(i + 2, cur))
            if not first:
                s_wait(nxt)

            def nx():
                dst_start(i + 1, nxt)
                si_wait(nxt)
                gh_start(i + 1, nxt)
            maybe(i + 1 < nchunk, nx)
            mul(cur)
            di_wait(cur)
            s_start(cur)

        src_start(0, 0)
        dst_start(0, 0)
        src_start(1, 1)
        si_wait(0)
        gh_start(0, 0)

        emit_step(0, 0, first=True)
        emit_step(1, 1, first=False)

        def pair(j, carry):
            i = j * 2
            emit_step(i, 0, first=False)
            emit_step(i + 1, 1, first=False)
            return carry

        lax.fori_loop(1, nchunk // 2, pair, 0)
        s_wait((nchunk - 1) % 2)

        plsc.subcore_barrier()
        pltpu.sync_copy(acc.at[pl.ds(sid * RPT, RPT)],
                        out_hbm.at[cid, pl.ds(sid * RPT, RPT)])

    return run(x, h, srca, dsta, zrows)


def kernel(node, edge, edge_index, W1, We1, be1, We2, be2, W2, b2, W3, b3):
    ei = edge_index.astype(jnp.int32)
    x = _node_dense(node, W1)
    zrows = jnp.zeros((ZR, F), jnp.float32)
    EH = E // 2
    EHP = EH + E_PAD
    pad_dst = N + (jnp.arange(E_PAD, dtype=jnp.int32) % (N_PAD - N))
    pad_src = jnp.arange(E_PAD, dtype=jnp.int32) % N
    src_a = jnp.concatenate([ei[1, :EH], pad_src])
    dst_a = jnp.concatenate([ei[0, :EH], pad_dst])
    src_b = jnp.concatenate([ei[1, EH:], pad_src])
    dst_b = jnp.concatenate([ei[0, EH:], pad_dst])
    h_a = _edge_mlp(edge, We1, be1, We2, be2, EHP, 0)
    p_a = _gather_mul_scatter(x, h_a, src_a, dst_a, zrows, EHP // NW, 80)
    h_b = _edge_mlp(edge, We1, be1, We2, be2, EHP, EH)
    p_b = _gather_mul_scatter(x, h_b, src_b, dst_b, zrows, EHP // NW, 80)
    return _out_mlp(node, p_a, p_b, W2, b2, W3, b3)


# speedup vs baseline: 2.4120x; 1.0259x over previous
import functools

import jax
import jax.numpy as jnp
from jax import lax
from jax.experimental import pallas as pl
from jax.experimental.pallas import tpu as pltpu
from jax.experimental.pallas import tpu_sc as plsc

N = 10000
E = 320000
F = 128
_LOG2 = 0.6931471805599453

NC = 2
NS = 16
NW = NC * NS
EPW = E // NW
C = 80
NCHUNK = EPW // C
N_PAD = 10240
RPT = N_PAD // NS
ZR = 128
NZ = RPT // ZR
E_PAD = 3840


def _ssp(x):
    return jax.nn.softplus(x) - _LOG2


def _node_dense(node, W1):
    BN = 1000

    def body(n_ref, w_ref, o_ref):
        o_ref[...] = jnp.dot(n_ref[...], w_ref[...],
                             preferred_element_type=jnp.float32)

    return pl.pallas_call(
        body,
        grid=(N // BN,),
        in_specs=[pl.BlockSpec((BN, F), lambda i: (i, 0)),
                  pl.BlockSpec((F, F), lambda i: (0, 0))],
        out_specs=pl.BlockSpec((BN, F), lambda i: (i, 0)),
        out_shape=jax.ShapeDtypeStruct((N, F), jnp.float32),
    )(node, W1)


def _edge_mlp(edge, We1, be1, We2, be2, nrows, row0, real_rows):
    BE = 640
    blk0 = row0 // BE
    nreal = real_rows // BE

    def body(e_ref, w1_ref, b1_ref, w2_ref, b2_ref, o_ref):
        @pl.when(pl.program_id(0) < nreal)
        def _():
            t = jnp.dot(e_ref[...], w1_ref[...],
                        preferred_element_type=jnp.float32)
            t = _ssp(t + b1_ref[...])
            o_ref[...] = jnp.dot(t, w2_ref[...],
                                 preferred_element_type=jnp.float32) + b2_ref[...]
        @pl.when(pl.program_id(0) >= nreal)
        def _():
            o_ref[...] = jnp.zeros((BE, F), jnp.float32)

    return pl.pallas_call(
        body,
        grid=(nrows // BE,),
        in_specs=[pl.BlockSpec((BE, F),
                               lambda i: (jnp.minimum(i, nreal - 1) + blk0, 0)),
                  pl.BlockSpec((F, F), lambda i: (0, 0)),
                  pl.BlockSpec((1, F), lambda i: (0, 0)),
                  pl.BlockSpec((F, F), lambda i: (0, 0)),
                  pl.BlockSpec((1, F), lambda i: (0, 0))],
        out_specs=pl.BlockSpec((BE, F), lambda i: (i, 0)),
        out_shape=jax.ShapeDtypeStruct((nrows, F), jnp.float32),
    )(edge, We1, be1.reshape(1, F), We2, be2.reshape(1, F))


def _out_mlp(node, p_a, p_b, W2, b2, W3, b3):
    BN = 1000

    def body(n_ref, p_ref, q_ref, w2_ref, b2_ref, w3_ref, b3_ref, o_ref):
        p = p_ref[0] + p_ref[1] + q_ref[0] + q_ref[1]
        t = _ssp(jnp.dot(p, w2_ref[...],
                         preferred_element_type=jnp.float32) + b2_ref[...])
        o_ref[...] = (n_ref[...] +
                      jnp.dot(t, w3_ref[...],
                              preferred_element_type=jnp.float32) + b3_ref[...])

    return pl.pallas_call(
        body,
        grid=(N // BN,),
        in_specs=[pl.BlockSpec((BN, F), lambda i: (i, 0)),
                  pl.BlockSpec((NC, BN, F), lambda i: (0, i, 0)),
                  pl.BlockSpec((NC, BN, F), lambda i: (0, i, 0)),
                  pl.BlockSpec((F, F), lambda i: (0, 0)),
                  pl.BlockSpec((1, F), lambda i: (0, 0)),
                  pl.BlockSpec((F, F), lambda i: (0, 0)),
                  pl.BlockSpec((1, F), lambda i: (0, 0))],
        out_specs=pl.BlockSpec((BN, F), lambda i: (i, 0)),
        out_shape=jax.ShapeDtypeStruct((N, F), jnp.float32),
    )(node, p_a, p_b, W2, b2.reshape(1, F), W3, b3.reshape(1, F))


def _gather_mul_scatter(x, h, srca, dsta, zrows, epw, c):
    nchunk = epw // c
    assert nchunk >= 4 and nchunk % 2 == 0 and c % 16 == 0 and nchunk * c == epw
    mesh = plsc.VectorSubcoreMesh(core_axis_name="c", subcore_axis_name="s")

    @functools.partial(
        pl.kernel,
        out_type=jax.ShapeDtypeStruct((NC, N_PAD, F), jnp.float32),
        mesh=mesh,
        scratch_types=[
            [pltpu.VMEM((c,), jnp.int32)] * 2,
            [pltpu.VMEM((c,), jnp.int32)] * 2,
            [pltpu.VMEM((c, F), jnp.float32)] * 2,
            [pltpu.VMEM((c, F), jnp.float32)] * 2,
            pltpu.VMEM_SHARED((N_PAD, F), jnp.float32),
            [pltpu.SemaphoreType.DMA] * 2,
            [pltpu.SemaphoreType.DMA] * 2,
            [pltpu.SemaphoreType.DMA] * 2,
            [pltpu.SemaphoreType.DMA] * 2,
        ],
    )
    def run(x_hbm, h_hbm, src_hbm, dst_hbm, z_hbm, out_hbm,
            srcv, dstv, xr, hr, acc, sem_si, sem_di, sem_gh, sem_s):
        cid = lax.axis_index("c")
        sid = lax.axis_index("s")

        for j in range(NZ):
            pltpu.sync_copy(z_hbm, acc.at[pl.ds(sid * RPT + j * ZR, ZR)])
        plsc.subcore_barrier()

        base = (cid * NS + sid) * epw

        def src_start(i, b):
            pltpu.async_copy(src_hbm.at[pl.ds(base + i * c, c)],
                             srcv[b], sem_si[b])

        def dst_start(i, b):
            pltpu.async_copy(dst_hbm.at[pl.ds(base + i * c, c)],
                             dstv[b], sem_di[b])

        def gh_start(i, b):
            pltpu.async_copy(x_hbm.at[srcv[b]], xr[b], sem_gh[b])
            pltpu.async_copy(h_hbm.at[pl.ds(base + i * c, c)], hr[b],
                             sem_gh[b])

        def gh_wait(b):
            pltpu.make_async_copy(x_hbm.at[srcv[b]], xr[b], sem_gh[b]).wait()
            pltpu.make_async_copy(h_hbm.at[pl.ds(base, c)], hr[b],
                                  sem_gh[b]).wait()

        def si_wait(b):
            pltpu.make_async_copy(src_hbm.at[pl.ds(base, c)], srcv[b],
                                  sem_si[b]).wait()

        def di_wait(b):
            pltpu.make_async_copy(dst_hbm.at[pl.ds(base, c)], dstv[b],
                                  sem_di[b]).wait()

        def s_start(b):
            pltpu.async_copy(xr[b], acc.at[dstv[b]], sem_s[b], add=True)

        def s_wait(b):
            pltpu.make_async_copy(xr[b], acc.at[dstv[b]], sem_s[b]).wait()

        def mul(b):
            def mrow(j, c2):
                for u in range(4):
                    r = j * 4 + u
                    for k in range(F // 16):
                        sl = pl.ds(k * 16, 16)
                        xr[b][r, sl] = xr[b][r, sl] * hr[b][r, sl]
                return c2
            lax.fori_loop(0, c // 4, mrow, 0)

        def maybe(cond, fn):
            if isinstance(cond, bool):
                if cond:
                    fn()
            else:
                pl.when(cond)(fn)

        def emit_step(i, cur, first):
            nxt = 1 - cur
            gh_wait(cur)
            maybe(i + 2 < nchunk, lambda: src_start(i + 2, cur))
            if not first:
                s_wait(nxt)

            def nx():
                dst_start(i + 1, nxt)
                si_wait(nxt)
                gh_start(i + 1, nxt)
            maybe(i + 1 < nchunk, nx)
            mul(cur)
            di_wait(cur)
            s_start(cur)

        src_start(0, 0)
        dst_start(0, 0)
        src_start(1, 1)
        si_wait(0)
        gh_start(0, 0)

        emit_step(0, 0, first=True)
        emit_step(1, 1, first=False)

        def pair(j, carry):
            i = j * 2
            emit_step(i, 0, first=False)
            emit_step(i + 1, 1, first=False)
            return carry

        lax.fori_loop(1, nchunk // 2, pair, 0)
        s_wait((nchunk - 1) % 2)

        plsc.subcore_barrier()
        pltpu.sync_copy(acc.at[pl.ds(sid * RPT, RPT)],
                        out_hbm.at[cid, pl.ds(sid * RPT, RPT)])

    return run(x, h, srca, dsta, zrows)


def kernel(node, edge, edge_index, W1, We1, be1, We2, be2, W2, b2, W3, b3):
    ei = edge_index.astype(jnp.int32)
    x = _node_dense(node, W1)
    zrows = jnp.zeros((ZR, F), jnp.float32)
    EA = 184320
    EB = E - EA
    EBP = 138240
    PADB = EBP - EB
    pad_dst = N + (jnp.arange(PADB, dtype=jnp.int32) % (N_PAD - N))
    pad_src = jnp.arange(PADB, dtype=jnp.int32) % N
    src_a = ei[1, :EA]
    dst_a = ei[0, :EA]
    src_b = jnp.concatenate([ei[1, EA:], pad_src])
    dst_b = jnp.concatenate([ei[0, EA:], pad_dst])
    h_a = _edge_mlp(edge, We1, be1, We2, be2, EA, 0, EA)
    p_a = _gather_mul_scatter(x, h_a, src_a, dst_a, zrows, EA // NW, 80)
    h_b = _edge_mlp(edge, We1, be1, We2, be2, EBP, EA, EB)
    p_b = _gather_mul_scatter(x, h_b, src_b, dst_b, zrows, EBP // NW, 80)
    return _out_mlp(node, p_a, p_b, W2, b2, W3, b3)
